# baseline (speedup 1.0000x reference)
import functools

import jax
import jax.numpy as jnp
from jax import lax
from jax.experimental import pallas as pl
from jax.experimental.pallas import tpu as pltpu
from jax.experimental.pallas import tpu_sc as plsc

NC = 2
NS = 16
L = 16
NW = NC * NS

ALPHA = 0.5
K_HOPS = 2

_MESH = plsc.VectorSubcoreMesh(core_axis_name="c", subcore_axis_name="s")

_CHUNK = 40
_NBUF = 6
_HDT = jnp.float32


def _make_counts_kernel(n, e):
    epw = e // NW

    @functools.partial(
        pl.kernel,
        out_type=jax.ShapeDtypeStruct((NW, 2, n), jnp.float32),
        mesh=_MESH,
        scratch_types=[
            pltpu.VMEM((epw,), jnp.int32),
            pltpu.VMEM((epw,), jnp.int32),
            pltpu.VMEM((n,), jnp.float32),
            pltpu.VMEM((n,), jnp.float32),
        ],
        compiler_params=pltpu.CompilerParams(needs_layout_passes=False),
    )
    def counts_kernel(src_hbm, dst_hbm, out_hbm, src_v, dst_v, cs_v, cd_v):
        c = lax.axis_index("c")
        s = lax.axis_index("s")
        wid = s * NC + c
        base = wid * epw
        pltpu.sync_copy(src_hbm.at[pl.ds(base, epw)], src_v)
        pltpu.sync_copy(dst_hbm.at[pl.ds(base, epw)], dst_v)

        zeros = jnp.zeros((L,), jnp.float32)

        @pl.loop(0, n // L)
        def _zero(i):
            cs_v[pl.ds(i * L, L)] = zeros
            cd_v[pl.ds(i * L, L)] = zeros

        ones = jnp.ones((L,), jnp.float32)

        @pl.loop(0, epw // L)
        def _count(i):
            si = src_v[pl.ds(i * L, L)]
            di = dst_v[pl.ds(i * L, L)]
            plsc.addupdate_scatter(cs_v, [si], ones)
            plsc.addupdate_scatter(cd_v, [di], ones)

        pltpu.sync_copy(cs_v, out_hbm.at[wid, 0])
        pltpu.sync_copy(cd_v, out_hbm.at[wid, 1])

    return counts_kernel


def _make_hop_kernel(n, e, d):
    epw = e // NW
    chunk = _CHUNK
    nchunk = epw // chunk
    nbuf = _NBUF
    zr = 8
    rpt = -(-n // (NS * zr)) * zr
    rpt_last = n - (NS - 1) * rpt
    assert rpt_last > 0 and rpt_last % zr == 0

    @functools.partial(
        pl.kernel,
        out_type=jax.ShapeDtypeStruct((NC, n, d), _HDT),
        mesh=_MESH,
        scratch_types=[
            pltpu.VMEM_SHARED((n, d), _HDT),
            pltpu.VMEM((epw,), jnp.int32),
            pltpu.VMEM((epw,), jnp.int32),
            pltpu.VMEM((nbuf, chunk, d), _HDT),
            pltpu.SemaphoreType.DMA((nbuf,)),
            pltpu.SemaphoreType.DMA((nbuf,)),
        ],
    )
    def hop_kernel(h_hbm, src_hbm, dst_hbm, out_hbm, acc_sh, src_v, dst_v,
                   rows_v, gsem, ssem):
        c = lax.axis_index("c")
        s = lax.axis_index("s")
        wid = s * NC + c
        last = s == NS - 1
        base_r = s * rpt

        zeros = jnp.zeros((L,), _HDT)

        @pl.loop(0, zr)
        def _zrow(i):
            @pl.loop(0, d // L)
            def _zcol(j):
                rows_v[0, i, pl.ds(j * L, L)] = zeros

        nblk = jnp.where(last, rpt_last // zr, rpt // zr)

        @pl.loop(0, nblk)
        def _zcopy(k):
            pltpu.sync_copy(rows_v.at[0, pl.ds(0, zr)],
                            acc_sh.at[pl.ds(base_r + k * zr, zr)])

        base = wid * epw
        pltpu.sync_copy(src_hbm.at[pl.ds(base, epw)], src_v)
        pltpu.sync_copy(dst_hbm.at[pl.ds(base, epw)], dst_v)
        plsc.subcore_barrier()

        def gather_desc(j, b):
            return pltpu.make_async_copy(
                h_hbm.at[src_v.at[pl.ds(j * chunk, chunk)]], rows_v.at[b],
                gsem.at[b])

        def scatter_desc(j, b):
            return pltpu.make_async_copy(
                rows_v.at[b], acc_sh.at[dst_v.at[pl.ds(j * chunk, chunk)]],
                ssem.at[b])

        for b in range(nbuf):
            gather_desc(b, b).start()

        full_iters = nchunk // nbuf

        @pl.loop(0, full_iters)
        def _edge(k):
            jbase = k * nbuf
            for b in range(nbuf):
                j = jbase + b
                gather_desc(j, b).wait()
                pltpu.async_copy(
                    rows_v.at[b],
                    acc_sh.at[dst_v.at[pl.ds(j * chunk, chunk)]],
                    ssem.at[b], add=True)
            for b in range(nbuf):
                j = jbase + b
                scatter_desc(j, b).wait()

                @pl.when(j + nbuf < nchunk)
                def _refill():
                    gather_desc(j + nbuf, b).start()

        for t in range(nchunk % nbuf):
            j = full_iters * nbuf + t
            b = j % nbuf
            gather_desc(j, b).wait()
            pltpu.async_copy(rows_v.at[b],
                             acc_sh.at[dst_v.at[pl.ds(j * chunk, chunk)]],
                             ssem.at[b], add=True)
            scatter_desc(j, b).wait()

        plsc.subcore_barrier()

        @pl.when(jnp.logical_not(last))
        def _dump():
            pltpu.sync_copy(acc_sh.at[pl.ds(base_r, rpt)],
                            out_hbm.at[c, pl.ds(base_r, rpt)])

        @pl.when(last)
        def _dump_last():
            pltpu.sync_copy(acc_sh.at[pl.ds((NS - 1) * rpt, rpt_last)],
                            out_hbm.at[c, pl.ds((NS - 1) * rpt, rpt_last)])

    return hop_kernel


def _tc_norms(cnt_ref, norms_ref):
    deg = jnp.maximum(jnp.sum(cnt_ref[...], axis=0), 1.0)
    norms_ref[...] = lax.rsqrt(deg)


def _tc_scale(feat_ref, ns_ref, out_ref):
    out_ref[...] = (feat_ref[...] * ns_ref[...]).astype(out_ref.dtype)


def _tc_blend_scale(p_ref, feat_ref, nd_ref, ns_ref, out_ref):
    agg = (p_ref[0].astype(jnp.float32) + p_ref[1].astype(jnp.float32))
    h = (1.0 - ALPHA) * (agg * nd_ref[...]) + ALPHA * feat_ref[...]
    out_ref[...] = (h * ns_ref[...]).astype(out_ref.dtype)


def _tc_final(p_ref, feat_ref, nd_ref, w1_ref, b1_ref, w2_ref, b2_ref,
              out_ref):
    agg = (p_ref[0].astype(jnp.float32) + p_ref[1].astype(jnp.float32))
    h = (1.0 - ALPHA) * (agg * nd_ref[...]) + ALPHA * feat_ref[...]
    h = jnp.dot(h, w1_ref[...], preferred_element_type=jnp.float32)
    h = jax.nn.relu(h + b1_ref[...])
    h = jnp.dot(h, w2_ref[...], preferred_element_type=jnp.float32)
    out_ref[...] = jax.nn.sigmoid(h + b2_ref[...])


def kernel(features, edge_index, W1, b1, W2, b2):
    n, d_in = features.shape
    e = edge_index.shape[1]
    d_out = W2.shape[1]

    src = edge_index[0]
    dst = edge_index[1]
    counts_part = _make_counts_kernel(n, e)(src, dst)

    norms = pl.pallas_call(
        _tc_norms,
        out_shape=jax.ShapeDtypeStruct((2, n), jnp.float32),
    )(counts_part)
    ns_col = norms[0].reshape(n, 1)
    nd_col = norms[1].reshape(n, 1)

    rb = 1000
    grid = n // rb
    assert n % rb == 0

    h_scaled = pl.pallas_call(
        _tc_scale,
        grid=(grid,),
        in_specs=[
            pl.BlockSpec((rb, d_in), lambda i: (i, 0)),
            pl.BlockSpec((rb, 1), lambda i: (i, 0)),
        ],
        out_specs=pl.BlockSpec((rb, d_in), lambda i: (i, 0)),
        out_shape=jax.ShapeDtypeStruct((n, d_in), _HDT),
    )(features, ns_col)

    hop = _make_hop_kernel(n, e, d_in)
    b1r = b1.reshape(1, -1)
    b2r = b2.reshape(1, -1)

    for _ in range(K_HOPS - 1):
        partials = hop(h_scaled, src, dst)
        h_scaled = pl.pallas_call(
            _tc_blend_scale,
            grid=(grid,),
            in_specs=[
                pl.BlockSpec((2, rb, d_in), lambda i: (0, i, 0)),
                pl.BlockSpec((rb, d_in), lambda i: (i, 0)),
                pl.BlockSpec((rb, 1), lambda i: (i, 0)),
                pl.BlockSpec((rb, 1), lambda i: (i, 0)),
            ],
            out_specs=pl.BlockSpec((rb, d_in), lambda i: (i, 0)),
            out_shape=jax.ShapeDtypeStruct((n, d_in), _HDT),
        )(partials, features, nd_col, ns_col)

    partials = hop(h_scaled, src, dst)
    out = pl.pallas_call(
        _tc_final,
        grid=(grid,),
        in_specs=[
            pl.BlockSpec((2, rb, d_in), lambda i: (0, i, 0)),
            pl.BlockSpec((rb, d_in), lambda i: (i, 0)),
            pl.BlockSpec((rb, 1), lambda i: (i, 0)),
            pl.BlockSpec((d_in, d_in), lambda i: (0, 0)),
            pl.BlockSpec((1, d_in), lambda i: (0, 0)),
            pl.BlockSpec((d_in, d_out), lambda i: (0, 0)),
            pl.BlockSpec((1, d_out), lambda i: (0, 0)),
        ],
        out_specs=pl.BlockSpec((rb, d_out), lambda i: (i, 0)),
        out_shape=jax.ShapeDtypeStruct((n, d_out), jnp.float32),
    )(partials, features, nd_col, W1, b1r, W2, b2r)
    return out


# speedup vs baseline: 1.0812x; 1.0812x over previous
import functools

import jax
import jax.numpy as jnp
from jax import lax
from jax.experimental import pallas as pl
from jax.experimental.pallas import tpu as pltpu
from jax.experimental.pallas import tpu_sc as plsc

NC = 2
NS = 16
L = 16
NW = NC * NS

ALPHA = 0.5
K_HOPS = 2

_MESH = plsc.VectorSubcoreMesh(core_axis_name="c", subcore_axis_name="s")

_CHUNK = 40
_NBUF = 6
_HDT = jnp.float32


def _make_counts_kernel(n, e):
    epw = e // NW

    @functools.partial(
        pl.kernel,
        out_type=jax.ShapeDtypeStruct((NW, 2, n), jnp.float32),
        mesh=_MESH,
        scratch_types=[
            pltpu.VMEM((epw,), jnp.int32),
            pltpu.VMEM((epw,), jnp.int32),
            pltpu.VMEM((n,), jnp.float32),
            pltpu.VMEM((n,), jnp.float32),
        ],
        compiler_params=pltpu.CompilerParams(needs_layout_passes=False),
    )
    def counts_kernel(src_hbm, dst_hbm, out_hbm, src_v, dst_v, cs_v, cd_v):
        c = lax.axis_index("c")
        s = lax.axis_index("s")
        wid = s * NC + c
        base = wid * epw
        pltpu.sync_copy(src_hbm.at[pl.ds(base, epw)], src_v)
        pltpu.sync_copy(dst_hbm.at[pl.ds(base, epw)], dst_v)

        zeros = jnp.zeros((L,), jnp.float32)

        @pl.loop(0, n // L)
        def _zero(i):
            cs_v[pl.ds(i * L, L)] = zeros
            cd_v[pl.ds(i * L, L)] = zeros

        ones = jnp.ones((L,), jnp.float32)

        @pl.loop(0, epw // L)
        def _count(i):
            si = src_v[pl.ds(i * L, L)]
            di = dst_v[pl.ds(i * L, L)]
            plsc.addupdate_scatter(cs_v, [si], ones)
            plsc.addupdate_scatter(cd_v, [di], ones)

        pltpu.sync_copy(cs_v, out_hbm.at[wid, 0])
        pltpu.sync_copy(cd_v, out_hbm.at[wid, 1])

    return counts_kernel


def _make_hop_kernel(n, e, d):
    epw = e // NW
    chunk = _CHUNK
    nchunk = epw // chunk
    nbuf = _NBUF
    zr = 8
    rpt = -(-n // (NS * zr)) * zr
    rpt_last = n - (NS - 1) * rpt
    assert rpt_last > 0 and rpt_last % zr == 0

    @functools.partial(
        pl.kernel,
        out_type=jax.ShapeDtypeStruct((NC, n, d), _HDT),
        mesh=_MESH,
        scratch_types=[
            pltpu.VMEM_SHARED((n, d), _HDT),
            pltpu.VMEM((epw,), jnp.int32),
            pltpu.VMEM((epw,), jnp.int32),
            pltpu.VMEM((nbuf, chunk, d), _HDT),
            pltpu.SemaphoreType.DMA((nbuf,)),
            pltpu.SemaphoreType.DMA((nbuf,)),
        ],
    )
    def hop_kernel(h_hbm, src_hbm, dst_hbm, out_hbm, acc_sh, src_v, dst_v,
                   rows_v, gsem, ssem):
        c = lax.axis_index("c")
        s = lax.axis_index("s")
        wid = s * NC + c
        last = s == NS - 1
        base_r = s * rpt

        zeros = jnp.zeros((L,), _HDT)

        @pl.loop(0, zr)
        def _zrow(i):
            @pl.loop(0, d // L)
            def _zcol(j):
                rows_v[0, i, pl.ds(j * L, L)] = zeros

        base = wid * epw
        pltpu.async_copy(src_hbm.at[pl.ds(base, epw)], src_v, gsem.at[0])
        pltpu.async_copy(dst_hbm.at[pl.ds(base, epw)], dst_v, gsem.at[1])

        nblk = jnp.where(last, rpt_last // zr, rpt // zr)

        @pl.loop(0, nblk)
        def _zcopy(k):
            pltpu.make_async_copy(rows_v.at[0, pl.ds(0, zr)],
                                  acc_sh.at[pl.ds(base_r + k * zr, zr)],
                                  ssem.at[0]).start()

        @pl.loop(0, nblk)
        def _zdrain(k):
            pltpu.make_async_copy(rows_v.at[0, pl.ds(0, zr)],
                                  acc_sh.at[pl.ds(base_r, zr)],
                                  ssem.at[0]).wait()

        pltpu.make_async_copy(src_hbm.at[pl.ds(base, epw)], src_v,
                              gsem.at[0]).wait()
        pltpu.make_async_copy(dst_hbm.at[pl.ds(base, epw)], dst_v,
                              gsem.at[1]).wait()
        plsc.subcore_barrier()

        def gather_desc(j, b):
            return pltpu.make_async_copy(
                h_hbm.at[src_v.at[pl.ds(j * chunk, chunk)]], rows_v.at[b],
                gsem.at[b])

        def scatter_desc(j, b):
            return pltpu.make_async_copy(
                rows_v.at[b], acc_sh.at[dst_v.at[pl.ds(j * chunk, chunk)]],
                ssem.at[b])

        for b in range(nbuf):
            gather_desc(b, b).start()

        full_iters = nchunk // nbuf

        @pl.loop(0, full_iters)
        def _edge(k):
            jbase = k * nbuf
            for b in range(nbuf):
                j = jbase + b
                gather_desc(j, b).wait()
                pltpu.async_copy(
                    rows_v.at[b],
                    acc_sh.at[dst_v.at[pl.ds(j * chunk, chunk)]],
                    ssem.at[b], add=True)
            for b in range(nbuf):
                j = jbase + b
                scatter_desc(j, b).wait()

                @pl.when(j + nbuf < nchunk)
                def _refill():
                    gather_desc(j + nbuf, b).start()

        for t in range(nchunk % nbuf):
            j = full_iters * nbuf + t
            b = j % nbuf
            gather_desc(j, b).wait()
            pltpu.async_copy(rows_v.at[b],
                             acc_sh.at[dst_v.at[pl.ds(j * chunk, chunk)]],
                             ssem.at[b], add=True)
            scatter_desc(j, b).wait()

        plsc.subcore_barrier()

        @pl.when(jnp.logical_not(last))
        def _dump():
            pltpu.sync_copy(acc_sh.at[pl.ds(base_r, rpt)],
                            out_hbm.at[c, pl.ds(base_r, rpt)])

        @pl.when(last)
        def _dump_last():
            pltpu.sync_copy(acc_sh.at[pl.ds((NS - 1) * rpt, rpt_last)],
                            out_hbm.at[c, pl.ds((NS - 1) * rpt, rpt_last)])

    return hop_kernel


def _tc_norm_scale(cnt_ref, feat_ref, nt_ref, h0_ref):
    deg = jnp.maximum(jnp.sum(cnt_ref[...], axis=0), 1.0)
    norms = lax.rsqrt(deg)
    nt = norms.T
    nt_ref[...] = nt
    h0_ref[...] = (feat_ref[...] * nt[:, 0:1]).astype(h0_ref.dtype)


def _tc_blend_scale(p_ref, feat_ref, nd_ref, ns_ref, out_ref):
    agg = (p_ref[0].astype(jnp.float32) + p_ref[1].astype(jnp.float32))
    h = (1.0 - ALPHA) * (agg * nd_ref[...]) + ALPHA * feat_ref[...]
    out_ref[...] = (h * ns_ref[...]).astype(out_ref.dtype)


def _tc_final(p_ref, feat_ref, nd_ref, w1_ref, b1_ref, w2_ref, b2_ref,
              out_ref):
    agg = (p_ref[0].astype(jnp.float32) + p_ref[1].astype(jnp.float32))
    h = (1.0 - ALPHA) * (agg * nd_ref[...]) + ALPHA * feat_ref[...]
    h = jnp.dot(h, w1_ref[...], preferred_element_type=jnp.float32)
    h = jax.nn.relu(h + b1_ref[...])
    h = jnp.dot(h, w2_ref[...], preferred_element_type=jnp.float32)
    out_ref[...] = jax.nn.sigmoid(h + b2_ref[...])


def kernel(features, edge_index, W1, b1, W2, b2):
    n, d_in = features.shape
    e = edge_index.shape[1]
    d_out = W2.shape[1]

    src = edge_index[0]
    dst = edge_index[1]
    counts_part = _make_counts_kernel(n, e)(src, dst)

    norms_t, h_scaled = pl.pallas_call(
        _tc_norm_scale,
        out_shape=[
            jax.ShapeDtypeStruct((n, 2), jnp.float32),
            jax.ShapeDtypeStruct((n, d_in), _HDT),
        ],
    )(counts_part, features)
    ns_col = norms_t[:, 0:1]
    nd_col = norms_t[:, 1:2]

    hop = _make_hop_kernel(n, e, d_in)
    b1r = b1.reshape(1, -1)
    b2r = b2.reshape(1, -1)

    for _ in range(K_HOPS - 1):
        partials = hop(h_scaled, src, dst)
        h_scaled = pl.pallas_call(
            _tc_blend_scale,
            out_shape=jax.ShapeDtypeStruct((n, d_in), _HDT),
        )(partials, features, nd_col, ns_col)

    partials = hop(h_scaled, src, dst)
    out = pl.pallas_call(
        _tc_final,
        out_shape=jax.ShapeDtypeStruct((n, d_out), jnp.float32),
    )(partials, features, nd_col, W1, b1r, W2, b2r)
    return out


# speedup vs baseline: 1.0812x; 1.0001x over previous
import functools

import jax
import jax.numpy as jnp
from jax import lax
from jax.experimental import pallas as pl
from jax.experimental.pallas import tpu as pltpu
from jax.experimental.pallas import tpu_sc as plsc

NC = 2
NS = 16
L = 16
NW = NC * NS

ALPHA = 0.5
K_HOPS = 2

_MESH = plsc.VectorSubcoreMesh(core_axis_name="c", subcore_axis_name="s")

_CHUNK = 40
_NBUF = 6
_HDT = jnp.float32


def _make_counts_kernel(n, e):
    epw = e // NW

    @functools.partial(
        pl.kernel,
        out_type=jax.ShapeDtypeStruct((NW, 2, n), jnp.float32),
        mesh=_MESH,
        scratch_types=[
            pltpu.VMEM((epw,), jnp.int32),
            pltpu.VMEM((epw,), jnp.int32),
            pltpu.VMEM((n,), jnp.float32),
            pltpu.VMEM((n,), jnp.float32),
        ],
        compiler_params=pltpu.CompilerParams(needs_layout_passes=False),
    )
    def counts_kernel(src_hbm, dst_hbm, out_hbm, src_v, dst_v, cs_v, cd_v):
        c = lax.axis_index("c")
        s = lax.axis_index("s")
        wid = s * NC + c
        base = wid * epw
        pltpu.sync_copy(src_hbm.at[pl.ds(base, epw)], src_v)
        pltpu.sync_copy(dst_hbm.at[pl.ds(base, epw)], dst_v)

        zeros = jnp.zeros((L,), jnp.float32)

        @pl.loop(0, n // L)
        def _zero(i):
            cs_v[pl.ds(i * L, L)] = zeros
            cd_v[pl.ds(i * L, L)] = zeros

        ones = jnp.ones((L,), jnp.float32)

        @pl.loop(0, epw // L, unroll=8)
        def _count(i):
            si = src_v[pl.ds(i * L, L)]
            di = dst_v[pl.ds(i * L, L)]
            plsc.addupdate_scatter(cs_v, [si], ones)
            plsc.addupdate_scatter(cd_v, [di], ones)

        pltpu.sync_copy(cs_v, out_hbm.at[wid, 0])
        pltpu.sync_copy(cd_v, out_hbm.at[wid, 1])

    return counts_kernel


def _make_hop_kernel(n, e, d):
    epw = e // NW
    chunk = _CHUNK
    nchunk = epw // chunk
    nbuf = _NBUF
    zr = 8
    rpt = -(-n // (NS * zr)) * zr
    rpt_last = n - (NS - 1) * rpt
    assert rpt_last > 0 and rpt_last % zr == 0

    @functools.partial(
        pl.kernel,
        out_type=jax.ShapeDtypeStruct((NC, n, d), _HDT),
        mesh=_MESH,
        scratch_types=[
            pltpu.VMEM_SHARED((n, d), _HDT),
            pltpu.VMEM((epw,), jnp.int32),
            pltpu.VMEM((epw,), jnp.int32),
            pltpu.VMEM((nbuf, chunk, d), _HDT),
            pltpu.SemaphoreType.DMA((nbuf,)),
            pltpu.SemaphoreType.DMA((nbuf,)),
        ],
    )
    def hop_kernel(h_hbm, src_hbm, dst_hbm, out_hbm, acc_sh, src_v, dst_v,
                   rows_v, gsem, ssem):
        c = lax.axis_index("c")
        s = lax.axis_index("s")
        wid = s * NC + c
        last = s == NS - 1
        base_r = s * rpt

        zeros = jnp.zeros((L,), _HDT)

        @pl.loop(0, zr)
        def _zrow(i):
            @pl.loop(0, d // L)
            def _zcol(j):
                rows_v[0, i, pl.ds(j * L, L)] = zeros

        base = wid * epw
        pltpu.async_copy(src_hbm.at[pl.ds(base, epw)], src_v, gsem.at[0])
        pltpu.async_copy(dst_hbm.at[pl.ds(base, epw)], dst_v, gsem.at[1])

        nblk = jnp.where(last, rpt_last // zr, rpt // zr)

        @pl.loop(0, nblk)
        def _zcopy(k):
            pltpu.make_async_copy(rows_v.at[0, pl.ds(0, zr)],
                                  acc_sh.at[pl.ds(base_r + k * zr, zr)],
                                  ssem.at[0]).start()

        @pl.loop(0, nblk)
        def _zdrain(k):
            pltpu.make_async_copy(rows_v.at[0, pl.ds(0, zr)],
                                  acc_sh.at[pl.ds(base_r, zr)],
                                  ssem.at[0]).wait()

        pltpu.make_async_copy(src_hbm.at[pl.ds(base, epw)], src_v,
                              gsem.at[0]).wait()
        pltpu.make_async_copy(dst_hbm.at[pl.ds(base, epw)], dst_v,
                              gsem.at[1]).wait()
        plsc.subcore_barrier()

        def gather_desc(j, b):
            return pltpu.make_async_copy(
                h_hbm.at[src_v.at[pl.ds(j * chunk, chunk)]], rows_v.at[b],
                gsem.at[b])

        def scatter_desc(j, b):
            return pltpu.make_async_copy(
                rows_v.at[b], acc_sh.at[dst_v.at[pl.ds(j * chunk, chunk)]],
                ssem.at[b])

        for b in range(nbuf):
            gather_desc(b, b).start()

        full_iters = nchunk // nbuf

        @pl.loop(0, full_iters)
        def _edge(k):
            jbase = k * nbuf
            for b in range(nbuf):
                j = jbase + b
                gather_desc(j, b).wait()
                pltpu.async_copy(
                    rows_v.at[b],
                    acc_sh.at[dst_v.at[pl.ds(j * chunk, chunk)]],
                    ssem.at[b], add=True)
            for b in range(nbuf):
                j = jbase + b
                scatter_desc(j, b).wait()

                @pl.when(j + nbuf < nchunk)
                def _refill():
                    gather_desc(j + nbuf, b).start()

        for t in range(nchunk % nbuf):
            j = full_iters * nbuf + t
            b = j % nbuf
            gather_desc(j, b).wait()
            pltpu.async_copy(rows_v.at[b],
                             acc_sh.at[dst_v.at[pl.ds(j * chunk, chunk)]],
                             ssem.at[b], add=True)
            scatter_desc(j, b).wait()

        plsc.subcore_barrier()

        @pl.when(jnp.logical_not(last))
        def _dump():
            pltpu.sync_copy(acc_sh.at[pl.ds(base_r, rpt)],
                            out_hbm.at[c, pl.ds(base_r, rpt)])

        @pl.when(last)
        def _dump_last():
            pltpu.sync_copy(acc_sh.at[pl.ds((NS - 1) * rpt, rpt_last)],
                            out_hbm.at[c, pl.ds((NS - 1) * rpt, rpt_last)])

    return hop_kernel


def _tc_norm_scale(cnt_ref, feat_ref, nt_ref, h0_ref):
    deg = jnp.maximum(jnp.sum(cnt_ref[...], axis=0), 1.0)
    norms = lax.rsqrt(deg)
    nt = norms.T
    nt_ref[...] = nt
    h0_ref[...] = (feat_ref[...] * nt[:, 0:1]).astype(h0_ref.dtype)


def _tc_blend_scale(p_ref, feat_ref, nd_ref, ns_ref, out_ref):
    agg = (p_ref[0].astype(jnp.float32) + p_ref[1].astype(jnp.float32))
    h = (1.0 - ALPHA) * (agg * nd_ref[...]) + ALPHA * feat_ref[...]
    out_ref[...] = (h * ns_ref[...]).astype(out_ref.dtype)


def _tc_final(p_ref, feat_ref, nd_ref, w1_ref, b1_ref, w2_ref, b2_ref,
              out_ref):
    agg = (p_ref[0].astype(jnp.float32) + p_ref[1].astype(jnp.float32))
    h = (1.0 - ALPHA) * (agg * nd_ref[...]) + ALPHA * feat_ref[...]
    h = jnp.dot(h, w1_ref[...], preferred_element_type=jnp.float32)
    h = jax.nn.relu(h + b1_ref[...])
    h = jnp.dot(h, w2_ref[...], preferred_element_type=jnp.float32)
    out_ref[...] = jax.nn.sigmoid(h + b2_ref[...])


def kernel(features, edge_index, W1, b1, W2, b2):
    n, d_in = features.shape
    e = edge_index.shape[1]
    d_out = W2.shape[1]

    src = edge_index[0]
    dst = edge_index[1]
    counts_part = _make_counts_kernel(n, e)(src, dst)

    norms_t, h_scaled = pl.pallas_call(
        _tc_norm_scale,
        out_shape=[
            jax.ShapeDtypeStruct((n, 2), jnp.float32),
            jax.ShapeDtypeStruct((n, d_in), _HDT),
        ],
    )(counts_part, features)
    ns_col = norms_t[:, 0:1]
    nd_col = norms_t[:, 1:2]

    hop = _make_hop_kernel(n, e, d_in)
    b1r = b1.reshape(1, -1)
    b2r = b2.reshape(1, -1)

    for _ in range(K_HOPS - 1):
        partials = hop(h_scaled, src, dst)
        h_scaled = pl.pallas_call(
            _tc_blend_scale,
            out_shape=jax.ShapeDtypeStruct((n, d_in), _HDT),
        )(partials, features, nd_col, ns_col)

    partials = hop(h_scaled, src, dst)
    out = pl.pallas_call(
        _tc_final,
        out_shape=jax.ShapeDtypeStruct((n, d_out), jnp.float32),
    )(partials, features, nd_col, W1, b1r, W2, b2r)
    return out


# speedup vs baseline: 1.0815x; 1.0003x over previous
import functools

import jax
import jax.numpy as jnp
from jax import lax
from jax.experimental import pallas as pl
from jax.experimental.pallas import tpu as pltpu
from jax.experimental.pallas import tpu_sc as plsc

NC = 2
NS = 16
L = 16
NW = NC * NS

ALPHA = 0.5
K_HOPS = 2

_MESH = plsc.VectorSubcoreMesh(core_axis_name="c", subcore_axis_name="s")

_CHUNK = 40
_NBUF = 6
_HDT = jnp.float32


def _make_counts_kernel(n, e):
    epw = e // NW

    @functools.partial(
        pl.kernel,
        out_type=jax.ShapeDtypeStruct((NW, 2, n), jnp.float32),
        mesh=_MESH,
        scratch_types=[
            pltpu.VMEM((epw,), jnp.int32),
            pltpu.VMEM((epw,), jnp.int32),
            pltpu.VMEM((n,), jnp.float32),
            pltpu.VMEM((n,), jnp.float32),
        ],
        compiler_params=pltpu.CompilerParams(needs_layout_passes=False,
                                             skip_device_barrier=True),
    )
    def counts_kernel(src_hbm, dst_hbm, out_hbm, src_v, dst_v, cs_v, cd_v):
        c = lax.axis_index("c")
        s = lax.axis_index("s")
        wid = s * NC + c
        base = wid * epw
        pltpu.sync_copy(src_hbm.at[pl.ds(base, epw)], src_v)
        pltpu.sync_copy(dst_hbm.at[pl.ds(base, epw)], dst_v)

        zeros = jnp.zeros((L,), jnp.float32)

        @pl.loop(0, n // L)
        def _zero(i):
            cs_v[pl.ds(i * L, L)] = zeros
            cd_v[pl.ds(i * L, L)] = zeros

        ones = jnp.ones((L,), jnp.float32)

        @pl.loop(0, epw // L, unroll=8)
        def _count(i):
            si = src_v[pl.ds(i * L, L)]
            di = dst_v[pl.ds(i * L, L)]
            plsc.addupdate_scatter(cs_v, [si], ones)
            plsc.addupdate_scatter(cd_v, [di], ones)

        pltpu.sync_copy(cs_v, out_hbm.at[wid, 0])
        pltpu.sync_copy(cd_v, out_hbm.at[wid, 1])

    return counts_kernel


def _make_hop_kernel(n, e, d):
    epw = e // NW
    chunk = _CHUNK
    nchunk = epw // chunk
    nbuf = _NBUF
    zr = 8
    rpt = -(-n // (NS * zr)) * zr
    rpt_last = n - (NS - 1) * rpt
    assert rpt_last > 0 and rpt_last % zr == 0

    @functools.partial(
        pl.kernel,
        out_type=jax.ShapeDtypeStruct((NC, n, d), _HDT),
        mesh=_MESH,
        scratch_types=[
            pltpu.VMEM_SHARED((n, d), _HDT),
            pltpu.VMEM((epw,), jnp.int32),
            pltpu.VMEM((epw,), jnp.int32),
            pltpu.VMEM((nbuf, chunk, d), _HDT),
            pltpu.SemaphoreType.DMA((nbuf,)),
            pltpu.SemaphoreType.DMA((nbuf,)),
        ],
        compiler_params=pltpu.CompilerParams(skip_device_barrier=True),
    )
    def hop_kernel(h_hbm, src_hbm, dst_hbm, out_hbm, acc_sh, src_v, dst_v,
                   rows_v, gsem, ssem):
        c = lax.axis_index("c")
        s = lax.axis_index("s")
        wid = s * NC + c
        last = s == NS - 1
        base_r = s * rpt

        zeros = jnp.zeros((L,), _HDT)

        @pl.loop(0, zr)
        def _zrow(i):
            @pl.loop(0, d // L)
            def _zcol(j):
                rows_v[0, i, pl.ds(j * L, L)] = zeros

        base = wid * epw
        pltpu.async_copy(src_hbm.at[pl.ds(base, epw)], src_v, gsem.at[0])
        pltpu.async_copy(dst_hbm.at[pl.ds(base, epw)], dst_v, gsem.at[1])

        nblk = jnp.where(last, rpt_last // zr, rpt // zr)

        @pl.loop(0, nblk)
        def _zcopy(k):
            pltpu.make_async_copy(rows_v.at[0, pl.ds(0, zr)],
                                  acc_sh.at[pl.ds(base_r + k * zr, zr)],
                                  ssem.at[0]).start()

        @pl.loop(0, nblk)
        def _zdrain(k):
            pltpu.make_async_copy(rows_v.at[0, pl.ds(0, zr)],
                                  acc_sh.at[pl.ds(base_r, zr)],
                                  ssem.at[0]).wait()

        pltpu.make_async_copy(src_hbm.at[pl.ds(base, epw)], src_v,
                              gsem.at[0]).wait()
        pltpu.make_async_copy(dst_hbm.at[pl.ds(base, epw)], dst_v,
                              gsem.at[1]).wait()
        plsc.subcore_barrier()

        def gather_desc(j, b):
            return pltpu.make_async_copy(
                h_hbm.at[src_v.at[pl.ds(j * chunk, chunk)]], rows_v.at[b],
                gsem.at[b])

        def scatter_desc(j, b):
            return pltpu.make_async_copy(
                rows_v.at[b], acc_sh.at[dst_v.at[pl.ds(j * chunk, chunk)]],
                ssem.at[b])

        for b in range(nbuf):
            gather_desc(b, b).start()

        full_iters = nchunk // nbuf

        @pl.loop(0, full_iters)
        def _edge(k):
            jbase = k * nbuf
            for b in range(nbuf):
                j = jbase + b
                gather_desc(j, b).wait()
                pltpu.async_copy(
                    rows_v.at[b],
                    acc_sh.at[dst_v.at[pl.ds(j * chunk, chunk)]],
                    ssem.at[b], add=True)
            for b in range(nbuf):
                j = jbase + b
                scatter_desc(j, b).wait()

                @pl.when(j + nbuf < nchunk)
                def _refill():
                    gather_desc(j + nbuf, b).start()

        for t in range(nchunk % nbuf):
            j = full_iters * nbuf + t
            b = j % nbuf
            gather_desc(j, b).wait()
            pltpu.async_copy(rows_v.at[b],
                             acc_sh.at[dst_v.at[pl.ds(j * chunk, chunk)]],
                             ssem.at[b], add=True)
            scatter_desc(j, b).wait()

        plsc.subcore_barrier()

        @pl.when(jnp.logical_not(last))
        def _dump():
            pltpu.sync_copy(acc_sh.at[pl.ds(base_r, rpt)],
                            out_hbm.at[c, pl.ds(base_r, rpt)])

        @pl.when(last)
        def _dump_last():
            pltpu.sync_copy(acc_sh.at[pl.ds((NS - 1) * rpt, rpt_last)],
                            out_hbm.at[c, pl.ds((NS - 1) * rpt, rpt_last)])

    return hop_kernel


def _tc_norm_scale(cnt_ref, feat_ref, nt_ref, h0_ref):
    deg = jnp.maximum(jnp.sum(cnt_ref[...], axis=0), 1.0)
    norms = lax.rsqrt(deg)
    nt = norms.T
    nt_ref[...] = nt
    h0_ref[...] = (feat_ref[...] * nt[:, 0:1]).astype(h0_ref.dtype)


def _tc_blend_scale(p_ref, feat_ref, nd_ref, ns_ref, out_ref):
    agg = (p_ref[0].astype(jnp.float32) + p_ref[1].astype(jnp.float32))
    h = (1.0 - ALPHA) * (agg * nd_ref[...]) + ALPHA * feat_ref[...]
    out_ref[...] = (h * ns_ref[...]).astype(out_ref.dtype)


def _tc_final(p_ref, feat_ref, nd_ref, w1_ref, b1_ref, w2_ref, b2_ref,
              out_ref):
    agg = (p_ref[0].astype(jnp.float32) + p_ref[1].astype(jnp.float32))
    h = (1.0 - ALPHA) * (agg * nd_ref[...]) + ALPHA * feat_ref[...]
    h = jnp.dot(h, w1_ref[...], preferred_element_type=jnp.float32)
    h = jax.nn.relu(h + b1_ref[...])
    h = jnp.dot(h, w2_ref[...], preferred_element_type=jnp.float32)
    out_ref[...] = jax.nn.sigmoid(h + b2_ref[...])


def kernel(features, edge_index, W1, b1, W2, b2):
    n, d_in = features.shape
    e = edge_index.shape[1]
    d_out = W2.shape[1]

    src = edge_index[0]
    dst = edge_index[1]
    counts_part = _make_counts_kernel(n, e)(src, dst)

    norms_t, h_scaled = pl.pallas_call(
        _tc_norm_scale,
        out_shape=[
            jax.ShapeDtypeStruct((n, 2), jnp.float32),
            jax.ShapeDtypeStruct((n, d_in), _HDT),
        ],
    )(counts_part, features)
    ns_col = norms_t[:, 0:1]
    nd_col = norms_t[:, 1:2]

    hop = _make_hop_kernel(n, e, d_in)
    b1r = b1.reshape(1, -1)
    b2r = b2.reshape(1, -1)

    for _ in range(K_HOPS - 1):
        partials = hop(h_scaled, src, dst)
        h_scaled = pl.pallas_call(
            _tc_blend_scale,
            out_shape=jax.ShapeDtypeStruct((n, d_in), _HDT),
        )(partials, features, nd_col, ns_col)

    partials = hop(h_scaled, src, dst)
    out = pl.pallas_call(
        _tc_final,
        out_shape=jax.ShapeDtypeStruct((n, d_out), jnp.float32),
    )(partials, features, nd_col, W1, b1r, W2, b2r)
    return out


# speedup vs baseline: 1.1214x; 1.0368x over previous
import functools

import jax
import jax.numpy as jnp
from jax import lax
from jax.experimental import pallas as pl
from jax.experimental.pallas import tpu as pltpu
from jax.experimental.pallas import tpu_sc as plsc

NC = 2
NS = 16
L = 16
NW = NC * NS

ALPHA = 0.5
K_HOPS = 2

_MESH = plsc.VectorSubcoreMesh(core_axis_name="c", subcore_axis_name="s")

_CHUNK = 40
_NBUF = 6
_HDT = jnp.float32


def _make_counts_kernel(n, e):
    epw = e // NW

    @functools.partial(
        pl.kernel,
        out_type=jax.ShapeDtypeStruct((NW, 2, n), jnp.float32),
        mesh=_MESH,
        scratch_types=[
            pltpu.VMEM((epw,), jnp.int32),
            pltpu.VMEM((epw,), jnp.int32),
            pltpu.VMEM((n,), jnp.float32),
            pltpu.VMEM((n,), jnp.float32),
        ],
        compiler_params=pltpu.CompilerParams(needs_layout_passes=False,
                                             skip_device_barrier=True),
    )
    def counts_kernel(ei_hbm, out_hbm, src_v, dst_v, cs_v, cd_v):
        c = lax.axis_index("c")
        s = lax.axis_index("s")
        wid = s * NC + c
        base = wid * epw
        pltpu.sync_copy(ei_hbm.at[pl.ds(base, epw)], src_v)
        pltpu.sync_copy(ei_hbm.at[pl.ds(e + base, epw)], dst_v)

        zeros = jnp.zeros((L,), jnp.float32)

        @pl.loop(0, n // L)
        def _zero(i):
            cs_v[pl.ds(i * L, L)] = zeros
            cd_v[pl.ds(i * L, L)] = zeros

        ones = jnp.ones((L,), jnp.float32)

        @pl.loop(0, epw // L, unroll=8)
        def _count(i):
            si = src_v[pl.ds(i * L, L)]
            di = dst_v[pl.ds(i * L, L)]
            plsc.addupdate_scatter(cs_v, [si], ones)
            plsc.addupdate_scatter(cd_v, [di], ones)

        pltpu.sync_copy(cs_v, out_hbm.at[wid, 0])
        pltpu.sync_copy(cd_v, out_hbm.at[wid, 1])

    return counts_kernel


def _make_hop_kernel(n, e, d):
    epw = e // NW
    chunk = _CHUNK
    nchunk = epw // chunk
    nbuf = _NBUF
    zr = 8
    rpt = -(-n // (NS * zr)) * zr
    rpt_last = n - (NS - 1) * rpt
    assert rpt_last > 0 and rpt_last % zr == 0

    @functools.partial(
        pl.kernel,
        out_type=jax.ShapeDtypeStruct((NC, n, d), _HDT),
        mesh=_MESH,
        scratch_types=[
            pltpu.VMEM_SHARED((n, d), _HDT),
            pltpu.VMEM((epw,), jnp.int32),
            pltpu.VMEM((epw,), jnp.int32),
            pltpu.VMEM((nbuf, chunk, d), _HDT),
            pltpu.SemaphoreType.DMA((nbuf,)),
            pltpu.SemaphoreType.DMA((nbuf,)),
        ],
        compiler_params=pltpu.CompilerParams(skip_device_barrier=True),
    )
    def hop_kernel(h_hbm, ei_hbm, out_hbm, acc_sh, src_v, dst_v,
                   rows_v, gsem, ssem):
        c = lax.axis_index("c")
        s = lax.axis_index("s")
        wid = s * NC + c
        last = s == NS - 1
        base_r = s * rpt

        zeros = jnp.zeros((L,), _HDT)

        @pl.loop(0, zr)
        def _zrow(i):
            @pl.loop(0, d // L)
            def _zcol(j):
                rows_v[0, i, pl.ds(j * L, L)] = zeros

        base = wid * epw
        pltpu.async_copy(ei_hbm.at[pl.ds(base, epw)], src_v, gsem.at[0])
        pltpu.async_copy(ei_hbm.at[pl.ds(e + base, epw)], dst_v, gsem.at[1])

        nblk = jnp.where(last, rpt_last // zr, rpt // zr)

        @pl.loop(0, nblk)
        def _zcopy(k):
            pltpu.make_async_copy(rows_v.at[0, pl.ds(0, zr)],
                                  acc_sh.at[pl.ds(base_r + k * zr, zr)],
                                  ssem.at[0]).start()

        @pl.loop(0, nblk)
        def _zdrain(k):
            pltpu.make_async_copy(rows_v.at[0, pl.ds(0, zr)],
                                  acc_sh.at[pl.ds(base_r, zr)],
                                  ssem.at[0]).wait()

        pltpu.make_async_copy(ei_hbm.at[pl.ds(base, epw)], src_v,
                              gsem.at[0]).wait()
        pltpu.make_async_copy(ei_hbm.at[pl.ds(e + base, epw)], dst_v,
                              gsem.at[1]).wait()
        plsc.subcore_barrier()

        def gather_desc(j, b):
            return pltpu.make_async_copy(
                h_hbm.at[src_v.at[pl.ds(j * chunk, chunk)]], rows_v.at[b],
                gsem.at[b])

        def scatter_desc(j, b):
            return pltpu.make_async_copy(
                rows_v.at[b], acc_sh.at[dst_v.at[pl.ds(j * chunk, chunk)]],
                ssem.at[b])

        for b in range(nbuf):
            gather_desc(b, b).start()

        full_iters = nchunk // nbuf

        @pl.loop(0, full_iters)
        def _edge(k):
            jbase = k * nbuf
            for b in range(nbuf):
                j = jbase + b
                gather_desc(j, b).wait()
                pltpu.async_copy(
                    rows_v.at[b],
                    acc_sh.at[dst_v.at[pl.ds(j * chunk, chunk)]],
                    ssem.at[b], add=True)
            for b in range(nbuf):
                j = jbase + b
                scatter_desc(j, b).wait()

                @pl.when(j + nbuf < nchunk)
                def _refill():
                    gather_desc(j + nbuf, b).start()

        for t in range(nchunk % nbuf):
            j = full_iters * nbuf + t
            b = j % nbuf
            gather_desc(j, b).wait()
            pltpu.async_copy(rows_v.at[b],
                             acc_sh.at[dst_v.at[pl.ds(j * chunk, chunk)]],
                             ssem.at[b], add=True)
            scatter_desc(j, b).wait()

        plsc.subcore_barrier()

        @pl.when(jnp.logical_not(last))
        def _dump():
            pltpu.sync_copy(acc_sh.at[pl.ds(base_r, rpt)],
                            out_hbm.at[c, pl.ds(base_r, rpt)])

        @pl.when(last)
        def _dump_last():
            pltpu.sync_copy(acc_sh.at[pl.ds((NS - 1) * rpt, rpt_last)],
                            out_hbm.at[c, pl.ds((NS - 1) * rpt, rpt_last)])

    return hop_kernel


def _tc_norm_scale(cnt_ref, feat_ref, nt_ref, h0_ref):
    deg = jnp.maximum(jnp.sum(cnt_ref[...], axis=0), 1.0)
    norms = lax.rsqrt(deg)
    nt = norms.T
    nt_ref[...] = nt
    h0_ref[...] = (feat_ref[...] * nt[:, 0:1]).astype(h0_ref.dtype)


def _tc_blend_scale(p_ref, feat_ref, nd_ref, ns_ref, out_ref):
    agg = (p_ref[0].astype(jnp.float32) + p_ref[1].astype(jnp.float32))
    h = (1.0 - ALPHA) * (agg * nd_ref[...]) + ALPHA * feat_ref[...]
    out_ref[...] = (h * ns_ref[...]).astype(out_ref.dtype)


def _tc_final(p_ref, feat_ref, nd_ref, w1_ref, b1_ref, w2_ref, b2_ref,
              out_ref):
    agg = (p_ref[0].astype(jnp.float32) + p_ref[1].astype(jnp.float32))
    h = (1.0 - ALPHA) * (agg * nd_ref[...]) + ALPHA * feat_ref[...]
    h = jnp.dot(h, w1_ref[...], preferred_element_type=jnp.float32)
    h = jax.nn.relu(h + b1_ref[...])
    h = jnp.dot(h, w2_ref[...], preferred_element_type=jnp.float32)
    out_ref[...] = jax.nn.sigmoid(h + b2_ref[...])


def kernel(features, edge_index, W1, b1, W2, b2):
    n, d_in = features.shape
    e = edge_index.shape[1]
    d_out = W2.shape[1]

    ei_flat = edge_index.reshape(2 * e)
    counts_part = _make_counts_kernel(n, e)(ei_flat)

    norms_t, h_scaled = pl.pallas_call(
        _tc_norm_scale,
        out_shape=[
            jax.ShapeDtypeStruct((n, 2), jnp.float32),
            jax.ShapeDtypeStruct((n, d_in), _HDT),
        ],
    )(counts_part, features)
    ns_col = norms_t[:, 0:1]
    nd_col = norms_t[:, 1:2]

    hop = _make_hop_kernel(n, e, d_in)
    b1r = b1.reshape(1, -1)
    b2r = b2.reshape(1, -1)

    for _ in range(K_HOPS - 1):
        partials = hop(h_scaled, ei_flat)
        h_scaled = pl.pallas_call(
            _tc_blend_scale,
            out_shape=jax.ShapeDtypeStruct((n, d_in), _HDT),
        )(partials, features, nd_col, ns_col)

    partials = hop(h_scaled, ei_flat)
    out = pl.pallas_call(
        _tc_final,
        out_shape=jax.ShapeDtypeStruct((n, d_out), jnp.float32),
    )(partials, features, nd_col, W1, b1r, W2, b2r)
    return out


# speedup vs baseline: 1.1328x; 1.0102x over previous
import functools

import jax
import jax.numpy as jnp
from jax import lax
from jax.experimental import pallas as pl
from jax.experimental.pallas import tpu as pltpu
from jax.experimental.pallas import tpu_sc as plsc

NC = 2
NS = 16
L = 16
NW = NC * NS

ALPHA = 0.5
K_HOPS = 2

_MESH = plsc.VectorSubcoreMesh(core_axis_name="c", subcore_axis_name="s")

_CHUNK = 40
_NBUF = 6
_HDT = jnp.float32


def _make_counts_kernel(n, e):
    epw = e // NW

    @functools.partial(
        pl.kernel,
        out_type=jax.ShapeDtypeStruct((NW, 2, n), jnp.float32),
        mesh=_MESH,
        scratch_types=[
            pltpu.VMEM((epw,), jnp.int32),
            pltpu.VMEM((epw,), jnp.int32),
            pltpu.VMEM((n,), jnp.float32),
            pltpu.VMEM((n,), jnp.float32),
        ],
        compiler_params=pltpu.CompilerParams(needs_layout_passes=False,
                                             skip_device_barrier=True),
    )
    def counts_kernel(ei_hbm, out_hbm, src_v, dst_v, cs_v, cd_v):
        c = lax.axis_index("c")
        s = lax.axis_index("s")
        wid = s * NC + c
        base = wid * epw
        pltpu.sync_copy(ei_hbm.at[pl.ds(base, epw)], src_v)
        pltpu.sync_copy(ei_hbm.at[pl.ds(e + base, epw)], dst_v)

        zeros = jnp.zeros((L,), jnp.float32)

        @pl.loop(0, n // L)
        def _zero(i):
            cs_v[pl.ds(i * L, L)] = zeros
            cd_v[pl.ds(i * L, L)] = zeros

        ones = jnp.ones((L,), jnp.float32)

        @pl.loop(0, epw // L, unroll=8)
        def _count(i):
            si = src_v[pl.ds(i * L, L)]
            di = dst_v[pl.ds(i * L, L)]
            plsc.addupdate_scatter(cs_v, [si], ones)
            plsc.addupdate_scatter(cd_v, [di], ones)

        pltpu.sync_copy(cs_v, out_hbm.at[wid, 0])
        pltpu.sync_copy(cd_v, out_hbm.at[wid, 1])

    return counts_kernel


def _make_hop_kernel(n, e, d):
    epw = e // NW
    chunk = _CHUNK
    nchunk = epw // chunk
    nbuf = _NBUF
    zr = 8
    rpt = -(-n // (NS * zr)) * zr
    rpt_last = n - (NS - 1) * rpt
    assert rpt_last > 0 and rpt_last % zr == 0

    @functools.partial(
        pl.kernel,
        out_type=jax.ShapeDtypeStruct((NC, n, d), _HDT),
        mesh=_MESH,
        scratch_types=[
            pltpu.VMEM_SHARED((n, d), _HDT),
            pltpu.VMEM((epw,), jnp.int32),
            pltpu.VMEM((epw,), jnp.int32),
            pltpu.VMEM((nbuf, chunk, d), _HDT),
            pltpu.SemaphoreType.DMA((nbuf,)),
            pltpu.SemaphoreType.DMA((nbuf,)),
        ],
        compiler_params=pltpu.CompilerParams(skip_device_barrier=True),
    )
    def hop_kernel(h_hbm, ei_hbm, out_hbm, acc_sh, src_v, dst_v,
                   rows_v, gsem, ssem):
        c = lax.axis_index("c")
        s = lax.axis_index("s")
        wid = s * NC + c
        last = s == NS - 1
        base_r = s * rpt

        zeros = jnp.zeros((L,), _HDT)

        @pl.loop(0, zr)
        def _zrow(i):
            @pl.loop(0, d // L)
            def _zcol(j):
                rows_v[0, i, pl.ds(j * L, L)] = zeros

        base = wid * epw
        pltpu.async_copy(ei_hbm.at[pl.ds(base, epw)], src_v, gsem.at[0])
        pltpu.async_copy(ei_hbm.at[pl.ds(e + base, epw)], dst_v, gsem.at[1])

        nblk = jnp.where(last, rpt_last // zr, rpt // zr)

        @pl.loop(0, nblk)
        def _zcopy(k):
            pltpu.make_async_copy(rows_v.at[0, pl.ds(0, zr)],
                                  acc_sh.at[pl.ds(base_r + k * zr, zr)],
                                  ssem.at[0]).start()

        @pl.loop(0, nblk)
        def _zdrain(k):
            pltpu.make_async_copy(rows_v.at[0, pl.ds(0, zr)],
                                  acc_sh.at[pl.ds(base_r, zr)],
                                  ssem.at[0]).wait()

        pltpu.make_async_copy(ei_hbm.at[pl.ds(base, epw)], src_v,
                              gsem.at[0]).wait()
        pltpu.make_async_copy(ei_hbm.at[pl.ds(e + base, epw)], dst_v,
                              gsem.at[1]).wait()
        plsc.subcore_barrier()

        def gather_desc(j, b):
            return pltpu.make_async_copy(
                h_hbm.at[src_v.at[pl.ds(j * chunk, chunk)]], rows_v.at[b],
                gsem.at[b])

        def scatter_desc(j, b):
            return pltpu.make_async_copy(
                rows_v.at[b], acc_sh.at[dst_v.at[pl.ds(j * chunk, chunk)]],
                ssem.at[b])

        for b in range(nbuf):
            gather_desc(b, b).start()

        full_iters = nchunk // nbuf

        @pl.loop(0, full_iters)
        def _edge(k):
            jbase = k * nbuf
            for b in range(nbuf):
                j = jbase + b
                gather_desc(j, b).wait()
                pltpu.async_copy(
                    rows_v.at[b],
                    acc_sh.at[dst_v.at[pl.ds(j * chunk, chunk)]],
                    ssem.at[b], add=True)
            for b in range(nbuf):
                j = jbase + b
                scatter_desc(j, b).wait()

                @pl.when(j + nbuf < nchunk)
                def _refill():
                    gather_desc(j + nbuf, b).start()

        for t in range(nchunk % nbuf):
            j = full_iters * nbuf + t
            b = j % nbuf
            gather_desc(j, b).wait()
            pltpu.async_copy(rows_v.at[b],
                             acc_sh.at[dst_v.at[pl.ds(j * chunk, chunk)]],
                             ssem.at[b], add=True)
            scatter_desc(j, b).wait()

        plsc.subcore_barrier()

        @pl.when(jnp.logical_not(last))
        def _dump():
            pltpu.sync_copy(acc_sh.at[pl.ds(base_r, rpt)],
                            out_hbm.at[c, pl.ds(base_r, rpt)])

        @pl.when(last)
        def _dump_last():
            pltpu.sync_copy(acc_sh.at[pl.ds((NS - 1) * rpt, rpt_last)],
                            out_hbm.at[c, pl.ds((NS - 1) * rpt, rpt_last)])

    return hop_kernel


def _tc_norm_scale(cnt_ref, feat_ref, nt_ref, h0_ref):
    deg = jnp.maximum(jnp.sum(cnt_ref[...], axis=0), 1.0)
    norms = lax.rsqrt(deg)
    nt = norms.T
    nt_ref[...] = nt
    h0_ref[...] = (feat_ref[...] * nt[:, 0:1]).astype(h0_ref.dtype)


def _tc_blend_scale(p_ref, feat_ref, nt_ref, out_ref):
    agg = (p_ref[0].astype(jnp.float32) + p_ref[1].astype(jnp.float32))
    h = (1.0 - ALPHA) * (agg * nt_ref[:, 1:2]) + ALPHA * feat_ref[...]
    out_ref[...] = (h * nt_ref[:, 0:1]).astype(out_ref.dtype)


def _tc_final(p_ref, feat_ref, nt_ref, w1_ref, b1_ref, w2_ref, b2_ref,
              out_ref):
    agg = (p_ref[0].astype(jnp.float32) + p_ref[1].astype(jnp.float32))
    h = (1.0 - ALPHA) * (agg * nt_ref[:, 1:2]) + ALPHA * feat_ref[...]
    h = jnp.dot(h, w1_ref[...], preferred_element_type=jnp.float32)
    h = jax.nn.relu(h + b1_ref[...])
    h = jnp.dot(h, w2_ref[...], preferred_element_type=jnp.float32)
    out_ref[...] = jax.nn.sigmoid(h + b2_ref[...])


def kernel(features, edge_index, W1, b1, W2, b2):
    n, d_in = features.shape
    e = edge_index.shape[1]
    d_out = W2.shape[1]

    ei_flat = edge_index.reshape(2 * e)
    counts_part = _make_counts_kernel(n, e)(ei_flat)

    norms_t, h_scaled = pl.pallas_call(
        _tc_norm_scale,
        out_shape=[
            jax.ShapeDtypeStruct((n, 2), jnp.float32),
            jax.ShapeDtypeStruct((n, d_in), _HDT),
        ],
    )(counts_part, features)

    hop = _make_hop_kernel(n, e, d_in)
    b1r = b1.reshape(1, -1)
    b2r = b2.reshape(1, -1)

    for _ in range(K_HOPS - 1):
        partials = hop(h_scaled, ei_flat)
        h_scaled = pl.pallas_call(
            _tc_blend_scale,
            out_shape=jax.ShapeDtypeStruct((n, d_in), _HDT),
        )(partials, features, norms_t)

    partials = hop(h_scaled, ei_flat)
    out = pl.pallas_call(
        _tc_final,
        out_shape=jax.ShapeDtypeStruct((n, d_out), jnp.float32),
    )(partials, features, norms_t, W1, b1r, W2, b2r)
    return out


# speedup vs baseline: 1.1356x; 1.0025x over previous
import functools

import jax
import jax.numpy as jnp
from jax import lax
from jax.experimental import pallas as pl
from jax.experimental.pallas import tpu as pltpu
from jax.experimental.pallas import tpu_sc as plsc

NC = 2
NS = 16
L = 16
NW = NC * NS

ALPHA = 0.5
K_HOPS = 2

_MESH = plsc.VectorSubcoreMesh(core_axis_name="c", subcore_axis_name="s")

_CHUNK = 40
_NBUF = 6
_HDT = jnp.float32


def _make_counts_kernel(n, e):
    epw = e // NW

    @functools.partial(
        pl.kernel,
        out_type=jax.ShapeDtypeStruct((NW, 2, n), jnp.float32),
        mesh=_MESH,
        scratch_types=[
            pltpu.VMEM((epw,), jnp.int32),
            pltpu.VMEM((epw,), jnp.int32),
            pltpu.VMEM((n,), jnp.float32),
            pltpu.VMEM((n,), jnp.float32),
        ],
        compiler_params=pltpu.CompilerParams(needs_layout_passes=False,
                                             skip_device_barrier=True),
    )
    def counts_kernel(ei_hbm, out_hbm, src_v, dst_v, cs_v, cd_v):
        c = lax.axis_index("c")
        s = lax.axis_index("s")
        wid = s * NC + c
        base = wid * epw
        pltpu.sync_copy(ei_hbm.at[pl.ds(base, epw)], src_v)
        pltpu.sync_copy(ei_hbm.at[pl.ds(e + base, epw)], dst_v)

        zeros = jnp.zeros((L,), jnp.float32)

        @pl.loop(0, n // L)
        def _zero(i):
            cs_v[pl.ds(i * L, L)] = zeros
            cd_v[pl.ds(i * L, L)] = zeros

        ones = jnp.ones((L,), jnp.float32)

        @pl.loop(0, epw // L, unroll=8)
        def _count(i):
            si = src_v[pl.ds(i * L, L)]
            di = dst_v[pl.ds(i * L, L)]
            plsc.addupdate_scatter(cs_v, [si], ones)
            plsc.addupdate_scatter(cd_v, [di], ones)

        pltpu.sync_copy(cs_v, out_hbm.at[wid, 0])
        pltpu.sync_copy(cd_v, out_hbm.at[wid, 1])

    return counts_kernel


def _make_hop_kernel(n, e, d):
    epw = e // NW
    chunk = _CHUNK
    nchunk = epw // chunk
    nbuf = _NBUF
    zr = 8
    rpt = -(-n // (NS * zr)) * zr
    rpt_last = n - (NS - 1) * rpt
    assert rpt_last > 0 and rpt_last % zr == 0

    @functools.partial(
        pl.kernel,
        out_type=jax.ShapeDtypeStruct((NC, n, d), _HDT),
        mesh=_MESH,
        scratch_types=[
            pltpu.VMEM_SHARED((n, d), _HDT),
            pltpu.VMEM((epw,), jnp.int32),
            pltpu.VMEM((epw,), jnp.int32),
            pltpu.VMEM((nbuf, chunk, d), _HDT),
            pltpu.SemaphoreType.DMA((nbuf,)),
            pltpu.SemaphoreType.DMA((nbuf,)),
        ],
        compiler_params=pltpu.CompilerParams(skip_device_barrier=True),
    )
    def hop_kernel(h_hbm, ei_hbm, out_hbm, acc_sh, src_v, dst_v,
                   rows_v, gsem, ssem):
        c = lax.axis_index("c")
        s = lax.axis_index("s")
        wid = s * NC + c
        last = s == NS - 1
        base_r = s * rpt

        zeros = jnp.zeros((L,), _HDT)

        @pl.loop(0, zr)
        def _zrow(i):
            @pl.loop(0, d // L)
            def _zcol(j):
                rows_v[0, i, pl.ds(j * L, L)] = zeros

        base = wid * epw
        pltpu.async_copy(ei_hbm.at[pl.ds(base, epw)], src_v, gsem.at[0])
        pltpu.async_copy(ei_hbm.at[pl.ds(e + base, epw)], dst_v, gsem.at[1])

        nblk = jnp.where(last, rpt_last // zr, rpt // zr)

        @pl.loop(0, nblk)
        def _zcopy(k):
            pltpu.make_async_copy(rows_v.at[0, pl.ds(0, zr)],
                                  acc_sh.at[pl.ds(base_r + k * zr, zr)],
                                  ssem.at[0]).start()

        @pl.loop(0, nblk)
        def _zdrain(k):
            pltpu.make_async_copy(rows_v.at[0, pl.ds(0, zr)],
                                  acc_sh.at[pl.ds(base_r, zr)],
                                  ssem.at[0]).wait()

        pltpu.make_async_copy(ei_hbm.at[pl.ds(base, epw)], src_v,
                              gsem.at[0]).wait()
        pltpu.make_async_copy(ei_hbm.at[pl.ds(e + base, epw)], dst_v,
                              gsem.at[1]).wait()
        plsc.subcore_barrier()

        def gather_desc(j, b):
            return pltpu.make_async_copy(
                h_hbm.at[src_v.at[pl.ds(j * chunk, chunk)]], rows_v.at[b],
                gsem.at[b])

        def scatter_desc(j, b):
            return pltpu.make_async_copy(
                rows_v.at[b], acc_sh.at[dst_v.at[pl.ds(j * chunk, chunk)]],
                ssem.at[b])

        for b in range(nbuf):
            gather_desc(b, b).start()

        full_iters = nchunk // nbuf

        @pl.loop(0, full_iters)
        def _edge(k):
            jbase = k * nbuf
            for b in range(nbuf):
                j = jbase + b
                gather_desc(j, b).wait()
                pltpu.async_copy(
                    rows_v.at[b],
                    acc_sh.at[dst_v.at[pl.ds(j * chunk, chunk)]],
                    ssem.at[b], add=True)
            for b in range(nbuf):
                j = jbase + b
                scatter_desc(j, b).wait()

                @pl.when(j + nbuf < nchunk)
                def _refill():
                    gather_desc(j + nbuf, b).start()

        for t in range(nchunk % nbuf):
            j = full_iters * nbuf + t
            b = j % nbuf
            gather_desc(j, b).wait()
            pltpu.async_copy(rows_v.at[b],
                             acc_sh.at[dst_v.at[pl.ds(j * chunk, chunk)]],
                             ssem.at[b], add=True)
            scatter_desc(j, b).wait()

        plsc.subcore_barrier()

        @pl.when(jnp.logical_not(last))
        def _dump():
            pltpu.sync_copy(acc_sh.at[pl.ds(base_r, rpt)],
                            out_hbm.at[c, pl.ds(base_r, rpt)])

        @pl.when(last)
        def _dump_last():
            pltpu.sync_copy(acc_sh.at[pl.ds((NS - 1) * rpt, rpt_last)],
                            out_hbm.at[c, pl.ds((NS - 1) * rpt, rpt_last)])

    return hop_kernel


def _tc_norm_scale(cnt_ref, feat_ref, nt_ref, h0_ref):
    deg = jnp.maximum(jnp.sum(cnt_ref[...], axis=0), 1.0)
    norms = lax.rsqrt(deg)
    nt = norms.T
    nt_ref[...] = nt
    h0_ref[...] = (feat_ref[...] * nt[:, 0:1]).astype(h0_ref.dtype)


def _tc_blend_scale(p_ref, feat_ref, nt_ref, out_ref):
    agg = (p_ref[0].astype(jnp.float32) + p_ref[1].astype(jnp.float32))
    h = (1.0 - ALPHA) * (agg * nt_ref[:, 1:2]) + ALPHA * feat_ref[...]
    out_ref[...] = (h * nt_ref[:, 0:1]).astype(out_ref.dtype)


def _tc_final(p_ref, feat_ref, nt_ref, w1_ref, b1_ref, w2_ref, b2_ref,
              out_ref):
    agg = (p_ref[0].astype(jnp.float32) + p_ref[1].astype(jnp.float32))
    h = (1.0 - ALPHA) * (agg * nt_ref[:, 1:2]) + ALPHA * feat_ref[...]
    h = jnp.dot(h, w1_ref[...], preferred_element_type=jnp.float32)
    h = jax.nn.relu(h + b1_ref[...])
    h = jnp.dot(h, w2_ref[...], preferred_element_type=jnp.float32)
    out_ref[...] = jax.nn.sigmoid(h + b2_ref[...])


def kernel(features, edge_index, W1, b1, W2, b2):
    n, d_in = features.shape
    e = edge_index.shape[1]
    d_out = W2.shape[1]

    ei_flat = edge_index.reshape(2 * e)
    counts_part = _make_counts_kernel(n, e)(ei_flat)

    norms_t, h_scaled = pl.pallas_call(
        _tc_norm_scale,
        out_shape=[
            jax.ShapeDtypeStruct((n, 2), jnp.float32),
            jax.ShapeDtypeStruct((n, d_in), _HDT),
        ],
    )(counts_part, features)

    hop = _make_hop_kernel(n, e, d_in)
    b1r = b1.reshape(1, -1)
    b2r = b2.reshape(1, -1)

    for _ in range(K_HOPS - 1):
        partials = hop(h_scaled, ei_flat)
        h_scaled = pl.pallas_call(
            _tc_blend_scale,
            grid=(n // 2000,),
            in_specs=[
                pl.BlockSpec((2, 2000, d_in), lambda i: (0, i, 0)),
                pl.BlockSpec((2000, d_in), lambda i: (i, 0)),
                pl.BlockSpec((2000, 2), lambda i: (i, 0)),
            ],
            out_specs=pl.BlockSpec((2000, d_in), lambda i: (i, 0)),
            out_shape=jax.ShapeDtypeStruct((n, d_in), _HDT),
        )(partials, features, norms_t)

    partials = hop(h_scaled, ei_flat)
    out = pl.pallas_call(
        _tc_final,
        grid=(n // 2000,),
        in_specs=[
            pl.BlockSpec((2, 2000, d_in), lambda i: (0, i, 0)),
            pl.BlockSpec((2000, d_in), lambda i: (i, 0)),
            pl.BlockSpec((2000, 2), lambda i: (i, 0)),
            pl.BlockSpec((d_in, d_in), lambda i: (0, 0)),
            pl.BlockSpec((1, d_in), lambda i: (0, 0)),
            pl.BlockSpec((d_in, d_out), lambda i: (0, 0)),
            pl.BlockSpec((1, d_out), lambda i: (0, 0)),
        ],
        out_specs=pl.BlockSpec((2000, d_out), lambda i: (i, 0)),
        out_shape=jax.ShapeDtypeStruct((n, d_out), jnp.float32),
    )(partials, features, norms_t, W1, b1r, W2, b2r)
    return out
